# baseline (speedup 1.0000x reference)
import jax
import jax.numpy as jnp
from jax.experimental import pallas as pl


def _matvec_kernel(inp_ref, mask_ref, out_ref):
    v = inp_ref[...]
    out_ref[...] = jnp.sum(mask_ref[...] * v[:, None, :], axis=-1)


def kernel(input, attention_mask):
    B, S = input.shape
    TS = 256
    grid = (S // TS,)
    return pl.pallas_call(
        _matvec_kernel,
        grid=grid,
        in_specs=[
            pl.BlockSpec((B, S), lambda i: (0, 0)),
            pl.BlockSpec((B, TS, S), lambda i: (0, i, 0)),
        ],
        out_specs=pl.BlockSpec((B, TS), lambda i: (0, i)),
        out_shape=jax.ShapeDtypeStruct((B, S), jnp.float32),
    )(input, attention_mask)
